# baseline (speedup 1.0000x reference)
import functools

import jax
import jax.numpy as jnp
from jax import lax
from jax.experimental import pallas as pl
from jax.experimental.pallas import tpu as pltpu
from jax.experimental.pallas import tpu_sc as plsc

N_NODES = 10000
FDIM = 128
FW = 144
NC, NS = 2, 16
CH = 128
IB = 4
ACC_N = 10112
ZR = ACC_N // NS
NQ = 8

_CHUNKS = [(o, min(CH, ZR - o)) for o in range(0, ZR, CH)]
NCH = len(_CHUNKS)


def _sc_segment_sums(x_aug, src_idx, dst_idx, acc_idx, rows_per_tile):
    R = rows_per_tile
    mesh = plsc.VectorSubcoreMesh(core_axis_name="c", subcore_axis_name="s")

    @functools.partial(
        pl.kernel,
        out_type=jax.ShapeDtypeStruct((NC * ACC_N, FW), jnp.float32),
        mesh=mesh,
        compiler_params=pltpu.CompilerParams(use_tc_tiling_on_sc=False),
        scratch_types=[
            pltpu.VMEM((IB, CH), jnp.int32),
            pltpu.VMEM((IB, CH), jnp.int32),
            pltpu.VMEM((NQ, CH), jnp.int32),
            pltpu.VMEM((2, CH, FW), jnp.float32),
            pltpu.VMEM_SHARED((ACC_N, FW), jnp.float32),
            pltpu.SemaphoreType.DMA,
            pltpu.SemaphoreType.DMA,
        ],
    )
    def agg(x_hbm, src_hbm, dst_hbm, aidx_hbm, zf_hbm,
            sum_hbm,
            src_v, dst_v, iidx_v, rows_v, acc_s, gsem, ssem):
        c = lax.axis_index("c")
        s = lax.axis_index("s")
        idx_base = pl.multiple_of((c * NS + s) * R, IB)
        out_base = pl.multiple_of(c * ACC_N + s * ZR, 8)
        pltpu.sync_copy(aidx_hbm.at[pl.ds(pl.multiple_of(s * NQ, 8), NQ)],
                        iidx_v)
        pltpu.sync_copy(zf_hbm, rows_v.at[0])
        for q in range(NCH):
            pltpu.sync_copy(rows_v.at[0], acc_s.at[iidx_v.at[q]])
        plsc.subcore_barrier()

        pltpu.sync_copy(src_hbm.at[pl.ds(idx_base, IB)], src_v)
        pltpu.async_copy(x_hbm.at[src_v.at[0]], rows_v.at[0], gsem)

        def body(p, carry):
            base = pl.multiple_of(idx_base + p * IB, IB)

            @pl.when(p > 0)
            def _():
                pltpu.make_async_copy(rows_v.at[(IB - 1) % 2],
                                      acc_s.at[dst_v.at[IB - 1]], ssem).wait()

            pltpu.sync_copy(dst_hbm.at[pl.ds(base, IB)], dst_v)
            for k in range(IB):
                buf = rows_v.at[k % 2]
                pltpu.make_async_copy(x_hbm.at[src_v.at[k]], buf, gsem).wait()
                if k > 0:
                    pltpu.make_async_copy(rows_v.at[(k - 1) % 2],
                                          acc_s.at[dst_v.at[k - 1]],
                                          ssem).wait()
                if k < IB - 1:
                    pltpu.async_copy(x_hbm.at[src_v.at[k + 1]],
                                     rows_v.at[(k + 1) % 2], gsem)
                else:
                    @pl.when(p + 1 < R // IB)
                    def _():
                        pltpu.sync_copy(src_hbm.at[pl.ds(base + IB, IB)],
                                        src_v)
                        pltpu.async_copy(x_hbm.at[src_v.at[0]],
                                         rows_v.at[0], gsem)
                pltpu.async_copy(buf, acc_s.at[dst_v.at[k]], ssem, add=True)
            return carry

        lax.fori_loop(0, R // IB, body, 0)
        pltpu.make_async_copy(rows_v.at[(IB - 1) % 2],
                              acc_s.at[dst_v.at[IB - 1]], ssem).wait()
        plsc.subcore_barrier()
        for q, (off, ln) in enumerate(_CHUNKS):
            pltpu.sync_copy(acc_s.at[iidx_v.at[q]], rows_v.at[0])
            pltpu.sync_copy(rows_v.at[0].at[pl.ds(0, ln)],
                            sum_hbm.at[pl.ds(out_base + off, ln)])

    zf = jnp.zeros((CH, FW), jnp.float32)
    return agg(x_aug, src_idx, dst_idx, acc_idx, zf)


def _tc_head(sum_gw, cnt_gw, x_gw, sum_pf, cnt_pf, x_pf,
             W_l_p2g, b_l_p2g, W_r_p2g, W_l_g2p, b_l_g2p, W_r_g2p,
             W_lin, b_lin, prelu_a):
    B = 1000
    grid = (N_NODES // B,)

    def body(sg_ref, cg_ref, xg_ref, sp_ref, cp_ref, xp_ref,
             wlp_ref, blp_ref, wrp_ref, wlg_ref, blg_ref, wrg_ref,
             wlin_ref, blin_ref, a_ref, out_pf_ref, gw_ref):
        cg = jnp.maximum(cg_ref[:, 0:1], 1.0)
        mean_gw = sg_ref[...] / cg
        og = (jnp.dot(mean_gw, wlp_ref[...], preferred_element_type=jnp.float32)
              + blp_ref[...]
              + jnp.dot(xg_ref[...], wrp_ref[...],
                        preferred_element_type=jnp.float32))
        og = jnp.maximum(og, 0.0)
        cp = jnp.maximum(cp_ref[:, 0:1], 1.0)
        mean_pf = sp_ref[...] / cp
        op = (jnp.dot(mean_pf, wlg_ref[...], preferred_element_type=jnp.float32)
              + blg_ref[...]
              + jnp.dot(xp_ref[...], wrg_ref[...],
                        preferred_element_type=jnp.float32))
        out_pf_ref[...] = jnp.maximum(op, 0.0)
        g = (jnp.dot(og, wlin_ref[...], preferred_element_type=jnp.float32)
             + blin_ref[...])
        gw_ref[...] = jnp.where(g >= 0.0, g, a_ref[0, 0] * g)

    row_spec = pl.BlockSpec((B, FDIM), lambda i: (i, 0))
    cnt_spec = pl.BlockSpec((B, 16), lambda i: (i, 0))
    full = lambda r, c: pl.BlockSpec((r, c), lambda i: (0, 0))
    out_pf, gw = pl.pallas_call(
        body,
        grid=grid,
        in_specs=[row_spec, cnt_spec, row_spec, row_spec, cnt_spec, row_spec,
                  full(FDIM, FDIM), full(1, FDIM), full(FDIM, FDIM),
                  full(FDIM, FDIM), full(1, FDIM), full(FDIM, FDIM),
                  full(FDIM, 1), full(1, 1), full(1, 1)],
        out_specs=[pl.BlockSpec((B, FDIM), lambda i: (i, 0)),
                   pl.BlockSpec((B, 1), lambda i: (i, 0))],
        out_shape=[jax.ShapeDtypeStruct((N_NODES, FDIM), jnp.float32),
                   jax.ShapeDtypeStruct((N_NODES, 1), jnp.float32)],
    )(sum_gw, cnt_gw, x_gw, sum_pf, cnt_pf, x_pf,
      W_l_p2g, b_l_p2g.reshape(1, FDIM), W_r_p2g,
      W_l_g2p, b_l_g2p.reshape(1, FDIM), W_r_g2p,
      W_lin, b_lin.reshape(1, 1), prelu_a.reshape(1, 1))
    return out_pf, gw


def kernel(x_pfas_sites, x_gw_wells, edge_index_p2g, edge_index_g2p,
           W_l_p2g, b_l_p2g, W_r_p2g, W_l_g2p, b_l_g2p, W_r_g2p,
           W_lin, b_lin, prelu_a):
    E = edge_index_p2g.shape[1]
    R = -(-E // (NS * CH * IB)) * IB
    e_pad = NS * CH * R - E

    x_all = jnp.concatenate([x_pfas_sites, x_gw_wells], axis=0)
    x_aug = jnp.concatenate(
        [x_all,
         jnp.ones((2 * N_NODES, 1), jnp.float32),
         jnp.zeros((2 * N_NODES, FW - FDIM - 1), jnp.float32)], axis=1)
    pad_src = (jnp.arange(e_pad, dtype=jnp.int32) * 997) % N_NODES
    pad_dst = N_NODES + (jnp.arange(e_pad, dtype=jnp.int32) % (ACC_N - N_NODES))
    src = jnp.concatenate([
        edge_index_p2g[0], pad_src,
        edge_index_g2p[0] + N_NODES, pad_src + N_NODES,
    ]).reshape(NC * NS * R, CH)
    dst = jnp.concatenate([
        edge_index_p2g[1], pad_dst,
        edge_index_g2p[1], pad_dst,
    ]).reshape(NC * NS * R, CH)

    chunk_rows = jnp.minimum(
        jnp.arange(NQ, dtype=jnp.int32)[:, None] * CH
        + jnp.arange(CH, dtype=jnp.int32)[None, :], ZR - 1)
    acc_idx = (jnp.arange(NS, dtype=jnp.int32)[:, None, None] * ZR
               + chunk_rows[None]).reshape(NS * NQ, CH)

    sums = _sc_segment_sums(x_aug, src, dst, acc_idx, R)
    sums = sums.reshape(NC, ACC_N, FW)

    return _tc_head(sums[0, :N_NODES, :FDIM], sums[0, :N_NODES, FDIM:],
                    x_gw_wells,
                    sums[1, :N_NODES, :FDIM], sums[1, :N_NODES, FDIM:],
                    x_pfas_sites,
                    W_l_p2g, b_l_p2g, W_r_p2g, W_l_g2p, b_l_g2p, W_r_g2p,
                    W_lin, b_lin, prelu_a)


# speedup vs baseline: 1.0013x; 1.0013x over previous
import functools

import jax
import jax.numpy as jnp
from jax import lax
from jax.experimental import pallas as pl
from jax.experimental.pallas import tpu as pltpu
from jax.experimental.pallas import tpu_sc as plsc

N_NODES = 10000
FDIM = 128
FW = 144
NC, NS = 2, 16
CH = 128
IB = 4
ACC_N = 10112
ZR = ACC_N // NS
NQ = 8

_CHUNKS = [(o, min(CH, ZR - o)) for o in range(0, ZR, CH)]
NCH = len(_CHUNKS)


def _sc_segment_sums(x_aug, src_idx, dst_idx, acc_idx, rows_per_tile):
    R = rows_per_tile
    mesh = plsc.VectorSubcoreMesh(core_axis_name="c", subcore_axis_name="s")

    @functools.partial(
        pl.kernel,
        out_type=jax.ShapeDtypeStruct((NC * ACC_N, FW), jnp.float32),
        mesh=mesh,
        compiler_params=pltpu.CompilerParams(use_tc_tiling_on_sc=False),
        scratch_types=[
            pltpu.VMEM((IB, CH), jnp.int32),
            pltpu.VMEM((IB, CH), jnp.int32),
            pltpu.VMEM((NQ, CH), jnp.int32),
            pltpu.VMEM((2, CH, FW), jnp.float32),
            pltpu.VMEM_SHARED((ACC_N, FW), jnp.float32),
            pltpu.SemaphoreType.DMA,
        ],
    )
    def agg(x_hbm, src_hbm, dst_hbm, aidx_hbm, zf_hbm,
            sum_hbm,
            src_v, dst_v, iidx_v, rows_v, acc_s, gsem):
        c = lax.axis_index("c")
        s = lax.axis_index("s")
        idx_base = pl.multiple_of((c * NS + s) * R, IB)
        out_base = pl.multiple_of(c * ACC_N + s * ZR, 8)
        pltpu.sync_copy(aidx_hbm.at[pl.ds(pl.multiple_of(s * NQ, 8), NQ)],
                        iidx_v)
        pltpu.sync_copy(zf_hbm, rows_v.at[0])
        for q in range(NCH):
            pltpu.sync_copy(rows_v.at[0], acc_s.at[iidx_v.at[q]])
        plsc.subcore_barrier()

        pltpu.sync_copy(src_hbm.at[pl.ds(idx_base, IB)], src_v)
        pltpu.async_copy(x_hbm.at[src_v.at[0]], rows_v.at[0], gsem)

        def body(p, carry):
            base = pl.multiple_of(idx_base + p * IB, IB)
            pltpu.sync_copy(dst_hbm.at[pl.ds(base, IB)], dst_v)
            for k in range(IB):
                buf = rows_v.at[k % 2]
                pltpu.make_async_copy(x_hbm.at[src_v.at[k]], buf, gsem).wait()
                if k < IB - 1:
                    pltpu.async_copy(x_hbm.at[src_v.at[k + 1]],
                                     rows_v.at[(k + 1) % 2], gsem)
                else:
                    @pl.when(p + 1 < R // IB)
                    def _():
                        pltpu.sync_copy(src_hbm.at[pl.ds(base + IB, IB)],
                                        src_v)
                        pltpu.async_copy(x_hbm.at[src_v.at[0]],
                                         rows_v.at[0], gsem)
                pltpu.sync_copy(buf, acc_s.at[dst_v.at[k]], add=True)
            return carry

        lax.fori_loop(0, R // IB, body, 0)
        plsc.subcore_barrier()
        for q, (off, ln) in enumerate(_CHUNKS):
            pltpu.sync_copy(acc_s.at[iidx_v.at[q]], rows_v.at[0])
            pltpu.sync_copy(rows_v.at[0].at[pl.ds(0, ln)],
                            sum_hbm.at[pl.ds(out_base + off, ln)])

    zf = jnp.zeros((CH, FW), jnp.float32)
    return agg(x_aug, src_idx, dst_idx, acc_idx, zf)


def _tc_head(sum_gw, cnt_gw, x_gw, sum_pf, cnt_pf, x_pf,
             W_l_p2g, b_l_p2g, W_r_p2g, W_l_g2p, b_l_g2p, W_r_g2p,
             W_lin, b_lin, prelu_a):
    B = 1000
    grid = (N_NODES // B,)

    def body(sg_ref, cg_ref, xg_ref, sp_ref, cp_ref, xp_ref,
             wlp_ref, blp_ref, wrp_ref, wlg_ref, blg_ref, wrg_ref,
             wlin_ref, blin_ref, a_ref, out_pf_ref, gw_ref):
        cg = jnp.maximum(cg_ref[:, 0:1], 1.0)
        mean_gw = sg_ref[...] / cg
        og = (jnp.dot(mean_gw, wlp_ref[...], preferred_element_type=jnp.float32)
              + blp_ref[...]
              + jnp.dot(xg_ref[...], wrp_ref[...],
                        preferred_element_type=jnp.float32))
        og = jnp.maximum(og, 0.0)
        cp = jnp.maximum(cp_ref[:, 0:1], 1.0)
        mean_pf = sp_ref[...] / cp
        op = (jnp.dot(mean_pf, wlg_ref[...], preferred_element_type=jnp.float32)
              + blg_ref[...]
              + jnp.dot(xp_ref[...], wrg_ref[...],
                        preferred_element_type=jnp.float32))
        out_pf_ref[...] = jnp.maximum(op, 0.0)
        g = (jnp.dot(og, wlin_ref[...], preferred_element_type=jnp.float32)
             + blin_ref[...])
        gw_ref[...] = jnp.where(g >= 0.0, g, a_ref[0, 0] * g)

    row_spec = pl.BlockSpec((B, FDIM), lambda i: (i, 0))
    cnt_spec = pl.BlockSpec((B, 16), lambda i: (i, 0))
    full = lambda r, c: pl.BlockSpec((r, c), lambda i: (0, 0))
    out_pf, gw = pl.pallas_call(
        body,
        grid=grid,
        in_specs=[row_spec, cnt_spec, row_spec, row_spec, cnt_spec, row_spec,
                  full(FDIM, FDIM), full(1, FDIM), full(FDIM, FDIM),
                  full(FDIM, FDIM), full(1, FDIM), full(FDIM, FDIM),
                  full(FDIM, 1), full(1, 1), full(1, 1)],
        out_specs=[pl.BlockSpec((B, FDIM), lambda i: (i, 0)),
                   pl.BlockSpec((B, 1), lambda i: (i, 0))],
        out_shape=[jax.ShapeDtypeStruct((N_NODES, FDIM), jnp.float32),
                   jax.ShapeDtypeStruct((N_NODES, 1), jnp.float32)],
    )(sum_gw, cnt_gw, x_gw, sum_pf, cnt_pf, x_pf,
      W_l_p2g, b_l_p2g.reshape(1, FDIM), W_r_p2g,
      W_l_g2p, b_l_g2p.reshape(1, FDIM), W_r_g2p,
      W_lin, b_lin.reshape(1, 1), prelu_a.reshape(1, 1))
    return out_pf, gw


def kernel(x_pfas_sites, x_gw_wells, edge_index_p2g, edge_index_g2p,
           W_l_p2g, b_l_p2g, W_r_p2g, W_l_g2p, b_l_g2p, W_r_g2p,
           W_lin, b_lin, prelu_a):
    E = edge_index_p2g.shape[1]
    R = -(-E // (NS * CH * IB)) * IB
    e_pad = NS * CH * R - E

    x_all = jnp.concatenate([x_pfas_sites, x_gw_wells], axis=0)
    x_aug = jnp.concatenate(
        [x_all,
         jnp.ones((2 * N_NODES, 1), jnp.float32),
         jnp.zeros((2 * N_NODES, FW - FDIM - 1), jnp.float32)], axis=1)
    pad_src = (jnp.arange(e_pad, dtype=jnp.int32) * 997) % N_NODES
    pad_dst = N_NODES + (jnp.arange(e_pad, dtype=jnp.int32) % (ACC_N - N_NODES))
    src = jnp.concatenate([
        edge_index_p2g[0], pad_src,
        edge_index_g2p[0] + N_NODES, pad_src + N_NODES,
    ]).reshape(NC * NS * R, CH)
    dst = jnp.concatenate([
        edge_index_p2g[1], pad_dst,
        edge_index_g2p[1], pad_dst,
    ]).reshape(NC * NS * R, CH)

    chunk_rows = jnp.minimum(
        jnp.arange(NQ, dtype=jnp.int32)[:, None] * CH
        + jnp.arange(CH, dtype=jnp.int32)[None, :], ZR - 1)
    acc_idx = (jnp.arange(NS, dtype=jnp.int32)[:, None, None] * ZR
               + chunk_rows[None]).reshape(NS * NQ, CH)

    sums = _sc_segment_sums(x_aug, src, dst, acc_idx, R)
    sums = sums.reshape(NC, ACC_N, FW)

    return _tc_head(sums[0, :N_NODES, :FDIM], sums[0, :N_NODES, FDIM:],
                    x_gw_wells,
                    sums[1, :N_NODES, :FDIM], sums[1, :N_NODES, FDIM:],
                    x_pfas_sites,
                    W_l_p2g, b_l_p2g, W_r_p2g, W_l_g2p, b_l_g2p, W_r_g2p,
                    W_lin, b_lin, prelu_a)
